# baseline (speedup 1.0000x reference)
import functools

import jax
import jax.numpy as jnp
from jax import lax
from jax.experimental import pallas as pl
from jax.experimental.pallas import tpu as pltpu
from jax.experimental.pallas import tpu_sc as plsc

_NUM_EMB = 512
_DIM = 32
_BETA = 0.25
_TOK = 64 * 1024
_BLK = 2048
_NW = 32
_PER_W = _TOK // _NW


def _vq_tc_body(x_ref, emb_ref, idx_ref, loss_ref):
    xb = x_ref[...]
    emb = emb_ref[...]
    sim2 = lax.dot_general(xb, emb + emb, (((1,), (0,)), ((), ())),
                           preferred_element_type=jnp.float32)
    x2 = jnp.sum(xb * xb, axis=1, keepdims=True)
    e2 = jnp.sum(emb * emb, axis=0, keepdims=True)
    ones_col = jnp.ones((_BLK, 1), jnp.float32)
    ones_row = jnp.ones((1, _NUM_EMB), jnp.float32)
    lhs = jnp.concatenate([x2, ones_col], axis=1)
    rhs = jnp.concatenate([ones_row, e2], axis=0)
    t = lax.dot_general(lhs, rhs, (((1,), (0,)), ((), ())),
                        preferred_element_type=jnp.float32)
    dist = t - sim2
    m = jnp.min(dist, axis=1, keepdims=True)
    fiota = lax.broadcasted_iota(jnp.int32, dist.shape, 1).astype(jnp.float32)
    idxf = jnp.min(jnp.where(dist == m, fiota, float(_NUM_EMB)), axis=1,
                   keepdims=True)
    idx_ref[...] = idxf.astype(jnp.int32)

    @pl.when(pl.program_id(0) == 0)
    def _():
        loss_ref[0, 0] = 0.0

    loss_ref[0, 0] += jnp.sum(m)


def _tc_stage(x_flat, emb):
    return pl.pallas_call(
        _vq_tc_body,
        grid=(_TOK // _BLK,),
        in_specs=[
            pl.BlockSpec((_BLK, _DIM), lambda i: (i, 0)),
            pl.BlockSpec((_DIM, _NUM_EMB), lambda i: (0, 0)),
        ],
        out_specs=[
            pl.BlockSpec((_BLK, 1), lambda i: (i, 0)),
            pl.BlockSpec((1, 1), lambda i: (0, 0), memory_space=pltpu.SMEM),
        ],
        out_shape=[
            jax.ShapeDtypeStruct((_TOK, 1), jnp.int32),
            jax.ShapeDtypeStruct((1, 1), jnp.float32),
        ],
        compiler_params=pltpu.CompilerParams(
            dimension_semantics=("arbitrary",)),
    )(x_flat, emb)


_JCH = 128
_NJ = _PER_W // _JCH
_SEQ = 1024
_ROWS_W = _PER_W // _SEQ


def _sc_gather_body(tab_ref, idx_ref, out_ref, tab_s, idx_v, rows_v, sem):
    sid = lax.axis_index("s")
    wid = sid * 2 + lax.axis_index("c")

    @pl.when(sid == 0)
    def _():
        pltpu.sync_copy(tab_ref, tab_s)

    pltpu.sync_copy(idx_ref.at[wid], idx_v)
    plsc.subcore_barrier()
    copies = [
        pltpu.async_copy(tab_s.at[idx_v.at[j]],
                         rows_v.at[j // 8, pl.ds((j % 8) * _JCH, _JCH)], sem)
        for j in range(_NJ)
    ]
    for c in copies:
        c.wait()
    pltpu.sync_copy(rows_v, out_ref.at[pl.ds(wid * _ROWS_W, _ROWS_W)])


@functools.cache
def _sc_gather():
    return pl.kernel(
        _sc_gather_body,
        out_type=jax.ShapeDtypeStruct((_TOK // _SEQ, _SEQ, _DIM), jnp.float32),
        mesh=plsc.VectorSubcoreMesh(core_axis_name="c", subcore_axis_name="s"),
        scratch_types=[
            pltpu.VMEM_SHARED((_NUM_EMB, _DIM), jnp.float32),
            pltpu.VMEM((_NJ, _JCH), jnp.int32),
            pltpu.VMEM((_ROWS_W, _SEQ, _DIM), jnp.float32),
            pltpu.SemaphoreType.DMA,
        ],
        compiler_params=pltpu.CompilerParams(
            use_tc_tiling_on_sc=False, needs_layout_passes=False),
    )


def kernel(x, embeddings):
    x_flat = x.reshape(_TOK, _DIM)
    idx, loss_sum = _tc_stage(x_flat, embeddings)
    quantized = _sc_gather()(embeddings.T, idx.reshape(_NW, _NJ, _JCH))
    mean_d = loss_sum[0, 0] / jnp.float32(_TOK * _DIM)
    loss = _BETA * mean_d + mean_d
    return quantized, loss


# speedup vs baseline: 1.0362x; 1.0362x over previous
import functools

import jax
import jax.numpy as jnp
from jax import lax
from jax.experimental import pallas as pl
from jax.experimental.pallas import tpu as pltpu
from jax.experimental.pallas import tpu_sc as plsc

_NUM_EMB = 512
_DIM = 32
_BETA = 0.25
_TOK = 64 * 1024
_BLK = 2048
_NW = 32
_PER_W = _TOK // _NW


def _vq_tc_body(x_ref, emb_ref, idx_ref, loss_ref):
    xb = x_ref[...]
    emb = emb_ref[...]
    sim2 = lax.dot_general(xb, emb + emb, (((1,), (0,)), ((), ())),
                           preferred_element_type=jnp.float32)
    x2 = jnp.sum(xb * xb, axis=1, keepdims=True)
    e2 = jnp.sum(emb * emb, axis=0, keepdims=True)
    dist = (x2 + e2) - sim2
    m = jnp.min(dist, axis=1, keepdims=True)
    fiota = lax.broadcasted_iota(jnp.int32, dist.shape, 1).astype(jnp.float32)
    idxf = jnp.min(jnp.where(dist == m, fiota, float(_NUM_EMB)), axis=1,
                   keepdims=True)
    idx_ref[...] = idxf.astype(jnp.int32)

    @pl.when(pl.program_id(0) == 0)
    def _():
        loss_ref[0, 0] = 0.0

    loss_ref[0, 0] += jnp.sum(m)


def _tc_stage(x_flat, emb):
    return pl.pallas_call(
        _vq_tc_body,
        grid=(_TOK // _BLK,),
        in_specs=[
            pl.BlockSpec((_BLK, _DIM), lambda i: (i, 0)),
            pl.BlockSpec((_DIM, _NUM_EMB), lambda i: (0, 0)),
        ],
        out_specs=[
            pl.BlockSpec((_BLK, 1), lambda i: (i, 0)),
            pl.BlockSpec((1, 1), lambda i: (0, 0), memory_space=pltpu.SMEM),
        ],
        out_shape=[
            jax.ShapeDtypeStruct((_TOK, 1), jnp.int32),
            jax.ShapeDtypeStruct((1, 1), jnp.float32),
        ],
        compiler_params=pltpu.CompilerParams(
            dimension_semantics=("arbitrary",)),
    )(x_flat, emb)


_JCH = 128
_NJ = _PER_W // _JCH
_SEQ = 1024
_ROWS_W = _PER_W // _SEQ


def _sc_gather_body(tab_ref, idx_ref, out_ref, tab_s, idx_v, rows_v, sem):
    sid = lax.axis_index("s")
    wid = sid * 2 + lax.axis_index("c")

    @pl.when(sid == 0)
    def _():
        pltpu.sync_copy(tab_ref, tab_s)

    pltpu.sync_copy(idx_ref.at[wid], idx_v)
    plsc.subcore_barrier()
    copies = [
        pltpu.async_copy(tab_s.at[idx_v.at[j]],
                         rows_v.at[j // 8, pl.ds((j % 8) * _JCH, _JCH)], sem)
        for j in range(_NJ)
    ]
    for c in copies:
        c.wait()
    pltpu.sync_copy(rows_v, out_ref.at[pl.ds(wid * _ROWS_W, _ROWS_W)])


@functools.cache
def _sc_gather():
    return pl.kernel(
        _sc_gather_body,
        out_type=jax.ShapeDtypeStruct((_TOK // _SEQ, _SEQ, _DIM), jnp.float32),
        mesh=plsc.VectorSubcoreMesh(core_axis_name="c", subcore_axis_name="s"),
        scratch_types=[
            pltpu.VMEM_SHARED((_NUM_EMB, _DIM), jnp.float32),
            pltpu.VMEM((_NJ, _JCH), jnp.int32),
            pltpu.VMEM((_ROWS_W, _SEQ, _DIM), jnp.float32),
            pltpu.SemaphoreType.DMA,
        ],
        compiler_params=pltpu.CompilerParams(
            use_tc_tiling_on_sc=False, needs_layout_passes=False),
    )


def kernel(x, embeddings):
    x_flat = x.reshape(_TOK, _DIM)
    idx, loss_sum = _tc_stage(x_flat, embeddings)
    quantized = _sc_gather()(embeddings.T, idx.reshape(_NW, _NJ, _JCH))
    mean_d = loss_sum[0, 0] / jnp.float32(_TOK * _DIM)
    loss = _BETA * mean_d + mean_d
    return quantized, loss


# speedup vs baseline: 1.0785x; 1.0408x over previous
import functools

import jax
import jax.numpy as jnp
from jax import lax
from jax.experimental import pallas as pl
from jax.experimental.pallas import tpu as pltpu
from jax.experimental.pallas import tpu_sc as plsc

_NUM_EMB = 512
_DIM = 32
_BETA = 0.25
_TOK = 64 * 1024
_BLK = 4096
_NW = 32
_PER_W = _TOK // _NW


def _vq_tc_body(x_ref, emb_ref, idx_ref, loss_ref):
    xb = x_ref[...]
    emb = emb_ref[...]
    sim2 = lax.dot_general(xb, emb + emb, (((1,), (0,)), ((), ())),
                           preferred_element_type=jnp.float32)
    x2 = jnp.sum(xb * xb, axis=1, keepdims=True)
    e2 = jnp.sum(emb * emb, axis=0, keepdims=True)
    dist = (x2 + e2) - sim2
    m = jnp.min(dist, axis=1, keepdims=True)
    fiota = lax.broadcasted_iota(jnp.int32, dist.shape, 1).astype(jnp.float32)
    idxf = jnp.min(jnp.where(dist == m, fiota, float(_NUM_EMB)), axis=1,
                   keepdims=True)
    idx_ref[...] = idxf.astype(jnp.int32)

    @pl.when(pl.program_id(0) == 0)
    def _():
        loss_ref[0, 0] = 0.0

    loss_ref[0, 0] += jnp.sum(m)


def _tc_stage(x_flat, emb):
    return pl.pallas_call(
        _vq_tc_body,
        grid=(_TOK // _BLK,),
        in_specs=[
            pl.BlockSpec((_BLK, _DIM), lambda i: (i, 0)),
            pl.BlockSpec((_DIM, _NUM_EMB), lambda i: (0, 0)),
        ],
        out_specs=[
            pl.BlockSpec((_BLK, 1), lambda i: (i, 0)),
            pl.BlockSpec((1, 1), lambda i: (0, 0), memory_space=pltpu.SMEM),
        ],
        out_shape=[
            jax.ShapeDtypeStruct((_TOK, 1), jnp.int32),
            jax.ShapeDtypeStruct((1, 1), jnp.float32),
        ],
        compiler_params=pltpu.CompilerParams(
            dimension_semantics=("arbitrary",)),
    )(x_flat, emb)


_JCH = 128
_NJ = _PER_W // _JCH
_SEQ = 1024
_ROWS_W = _PER_W // _SEQ


def _sc_gather_body(tab_ref, idx_ref, out_ref, tab_s, idx_v, rows_v, sem):
    sid = lax.axis_index("s")
    wid = sid * 2 + lax.axis_index("c")

    @pl.when(sid == 0)
    def _():
        pltpu.sync_copy(tab_ref, tab_s)

    pltpu.sync_copy(idx_ref.at[wid], idx_v)
    plsc.subcore_barrier()
    copies = [
        pltpu.async_copy(tab_s.at[idx_v.at[j]],
                         rows_v.at[j // 8, pl.ds((j % 8) * _JCH, _JCH)], sem)
        for j in range(_NJ)
    ]
    for c in copies:
        c.wait()
    pltpu.sync_copy(rows_v, out_ref.at[pl.ds(wid * _ROWS_W, _ROWS_W)])


@functools.cache
def _sc_gather():
    return pl.kernel(
        _sc_gather_body,
        out_type=jax.ShapeDtypeStruct((_TOK // _SEQ, _SEQ, _DIM), jnp.float32),
        mesh=plsc.VectorSubcoreMesh(core_axis_name="c", subcore_axis_name="s"),
        scratch_types=[
            pltpu.VMEM_SHARED((_NUM_EMB, _DIM), jnp.float32),
            pltpu.VMEM((_NJ, _JCH), jnp.int32),
            pltpu.VMEM((_ROWS_W, _SEQ, _DIM), jnp.float32),
            pltpu.SemaphoreType.DMA,
        ],
        compiler_params=pltpu.CompilerParams(
            use_tc_tiling_on_sc=False, needs_layout_passes=False),
    )


def kernel(x, embeddings):
    x_flat = x.reshape(_TOK, _DIM)
    idx, loss_sum = _tc_stage(x_flat, embeddings)
    quantized = _sc_gather()(embeddings.T, idx.reshape(_NW, _NJ, _JCH))
    mean_d = loss_sum[0, 0] / jnp.float32(_TOK * _DIM)
    loss = _BETA * mean_d + mean_d
    return quantized, loss
